# baseline (speedup 1.0000x reference)
import functools

import jax
import jax.numpy as jnp
from jax import lax
from jax.experimental import pallas as pl
from jax.experimental.pallas import tpu as pltpu
from jax.experimental.pallas import tpu_sc as plsc

F32 = jnp.float32
NC, NS, L = 2, 16, 16
NW = NC * NS
CHUNK = 64


def _mesh():
    return plsc.VectorSubcoreMesh(core_axis_name="c", subcore_axis_name="s",
                                  num_cores=NC, num_subcores=NS)


def _worker_ids():
    c = lax.axis_index("c")
    s = lax.axis_index("s")
    return c, s, c * NS + s


def _fill(ref, start, n, value):
    v = jnp.full((L,), value, F32)
    for k in range(n // L):
        ref[pl.ds(start + k * L, L)] = v


def _deg_body(n0, n1, rps, dst_hbm, out_hbm, didx, ones, zb, acc):
    c, s, w = _worker_ids()
    nloc = jnp.where(c == 0, n0, n1)
    pltpu.sync_copy(dst_hbm.at[w], didx)
    _fill(ones, 0, CHUNK, 1.0)
    _fill(zb, 0, rps, 0.0)
    pltpu.sync_copy(zb, acc.at[pl.ds(s * rps, rps)])
    plsc.subcore_barrier()

    def body(j, carry):
        pltpu.sync_copy(ones, acc.at[didx.at[j]], add=True)
        return carry

    lax.fori_loop(0, nloc, body, 0)
    plsc.subcore_barrier()
    pltpu.sync_copy(acc.at[pl.ds(s * rps, rps)], out_hbm.at[c, pl.ds(s * rps, rps)])


def _deg_call(dstp, npad, n0, n1):
    rps = npad // NS
    nmax = max(n0, n1)
    fn = pl.kernel(
        functools.partial(_deg_body, n0, n1, rps),
        out_type=jax.ShapeDtypeStruct((NC, npad), F32),
        mesh=_mesh(),
        scratch_types=[
            pltpu.VMEM((nmax, CHUNK), jnp.int32),
            pltpu.VMEM((CHUNK,), F32),
            pltpu.VMEM((rps,), F32),
            pltpu.VMEM_SHARED((npad,), F32),
        ],
    )
    return fn(dstp)


def _prop_body(n0, n1, rps, d, hn_hbm, src_hbm, dst_hbm, out_hbm,
               sidx, dbuf, rbuf, acc,
               gs0, gs1, gs2, gs3, is0, is1, is2, is3):
    c, s, w = _worker_ids()
    nloc = jnp.where(c == 0, n0, n1)
    gsems = (gs0, gs1, gs2, gs3)
    isems = (is0, is1, is2, is3)
    pltpu.sync_copy(src_hbm.at[w], sidx)

    zv = jnp.zeros((L,), F32)
    zrows = CHUNK

    def zrow(r, carry):
        for k in range(d // L):
            rbuf[0, r, pl.ds(k * L, L)] = zv
        return carry

    lax.fori_loop(0, zrows, zrow, 0)
    pltpu.sync_copy(rbuf.at[0, pl.ds(0, zrows), :],
                    acc.at[pl.ds(s * rps, zrows), :])
    plsc.subcore_barrier()

    def dload(j, q):
        return pltpu.async_copy(dst_hbm.at[w, j], dbuf.at[q], isems[q])

    def dload_wait(j, q):
        pltpu.make_async_copy(dst_hbm.at[w, j], dbuf.at[q], isems[q]).wait()

    def gather(j, q):
        idx = sidx.at[pl.ds(j * CHUNK, CHUNK)]
        return pltpu.async_copy(hn_hbm.at[idx], rbuf.at[q], gsems[q])

    def gather_wait(j, q):
        idx = sidx.at[pl.ds(j * CHUNK, CHUNK)]
        pltpu.make_async_copy(hn_hbm.at[idx], rbuf.at[q], gsems[q]).wait()

    for q in range(4):
        dload(q, q)
        gather(q, q)

    def body(t, carry):
        jb = 4 * t
        for q in range(4):
            j = jb + q
            dload_wait(j, q)
            gather_wait(j, q)
            pltpu.sync_copy(rbuf.at[q], acc.at[dbuf.at[q]], add=True)

            @pl.when(j + 4 < nloc)
            def _():
                dload(j + 4, q)
                gather(j + 4, q)

        return carry

    lax.fori_loop(0, nloc // 4, body, 0)
    plsc.subcore_barrier()
    pltpu.sync_copy(acc.at[pl.ds(s * rps, rps), :],
                    out_hbm.at[c, pl.ds(s * rps, rps), :])


def _prop_call(hn, srcp, dstp, npad, n0, n1, d):
    rps = npad // NS
    nmax = max(n0, n1)
    fn = pl.kernel(
        functools.partial(_prop_body, n0, n1, rps, d),
        out_type=jax.ShapeDtypeStruct((NC, npad, d), F32),
        mesh=_mesh(),
        scratch_types=[
            pltpu.VMEM((nmax * CHUNK,), jnp.int32),
            pltpu.VMEM((4, CHUNK), jnp.int32),
            pltpu.VMEM((4, CHUNK, d), F32),
            pltpu.VMEM_SHARED((npad, d), F32),
        ] + [pltpu.SemaphoreType.DMA] * 8,
    )
    return fn(hn, srcp, dstp)


def _sprop_body(n0, n1, rps, y_hbm, src_hbm, dst_hbm, out_hbm,
                sidx, dbuf, vbuf, acc,
                gs0, gs1, gs2, gs3, is0, is1, is2, is3):
    c, s, w = _worker_ids()
    nloc = jnp.where(c == 0, n0, n1)
    gsems = (gs0, gs1, gs2, gs3)
    isems = (is0, is1, is2, is3)
    pltpu.sync_copy(src_hbm.at[w], sidx)
    _fill(vbuf.at[0], 0, CHUNK, 0.0)
    for t in range(rps // CHUNK):
        pltpu.sync_copy(vbuf.at[0], acc.at[pl.ds(s * rps + t * CHUNK, CHUNK)])
    plsc.subcore_barrier()

    def dload(j, q):
        return pltpu.async_copy(dst_hbm.at[w, j], dbuf.at[q], isems[q])

    def dload_wait(j, q):
        pltpu.make_async_copy(dst_hbm.at[w, j], dbuf.at[q], isems[q]).wait()

    def gather(j, q):
        idx = sidx.at[pl.ds(j * CHUNK, CHUNK)]
        return pltpu.async_copy(y_hbm.at[idx], vbuf.at[q], gsems[q])

    def gather_wait(j, q):
        idx = sidx.at[pl.ds(j * CHUNK, CHUNK)]
        pltpu.make_async_copy(y_hbm.at[idx], vbuf.at[q], gsems[q]).wait()

    for q in range(4):
        dload(q, q)
        gather(q, q)

    def body(t, carry):
        jb = 4 * t
        for q in range(4):
            j = jb + q
            dload_wait(j, q)
            gather_wait(j, q)
            pltpu.sync_copy(vbuf.at[q], acc.at[dbuf.at[q]], add=True)

            @pl.when(j + 4 < nloc)
            def _():
                dload(j + 4, q)
                gather(j + 4, q)

        return carry

    lax.fori_loop(0, nloc // 4, body, 0)
    plsc.subcore_barrier()
    pltpu.sync_copy(acc.at[pl.ds(s * rps, rps)], out_hbm.at[c, pl.ds(s * rps, rps)])


def _sprop_call(y, srcp, dstp, npad, n0, n1):
    rps = npad // NS
    nmax = max(n0, n1)
    fn = pl.kernel(
        functools.partial(_sprop_body, n0, n1, rps),
        out_type=jax.ShapeDtypeStruct((NC, npad), F32),
        mesh=_mesh(),
        scratch_types=[
            pltpu.VMEM((nmax * CHUNK,), jnp.int32),
            pltpu.VMEM((4, CHUNK), jnp.int32),
            pltpu.VMEM((4, CHUNK), F32),
            pltpu.VMEM_SHARED((npad,), F32),
        ] + [pltpu.SemaphoreType.DMA] * 8,
    )
    return fn(y, srcp, dstp)


def _tc_prep_body(deg_ref, x_ref, w_ref, hn_ref, dis_ref):
    dis = lax.rsqrt(deg_ref[0] + deg_ref[1] + 1.0)
    dis_ref[...] = dis
    hn_ref[...] = jnp.dot(x_ref[...], w_ref[...],
                          preferred_element_type=F32) * dis


def _tc_prep_call(deg3, xpad, w1, npad, d, h, br=1024):
    g = npad // br
    return pl.pallas_call(
        _tc_prep_body,
        grid=(g,),
        in_specs=[
            pl.BlockSpec((NC, br, 1), lambda i: (0, i, 0)),
            pl.BlockSpec((br, d), lambda i: (i, 0)),
            pl.BlockSpec((d, h), lambda i: (0, 0)),
        ],
        out_specs=[
            pl.BlockSpec((br, h), lambda i: (i, 0)),
            pl.BlockSpec((br, 1), lambda i: (i, 0)),
        ],
        out_shape=[
            jax.ShapeDtypeStruct((npad, h), F32),
            jax.ShapeDtypeStruct((npad, 1), F32),
        ],
    )(deg3, xpad, w1)


def _tc_layer_body(acc_ref, hn_ref, dis_ref, b_ref, w_ref, out_ref):
    a = acc_ref[0] + acc_ref[1] + hn_ref[...]
    hact = jnp.maximum(a * dis_ref[...] + b_ref[...], 0.0)
    out_ref[...] = jnp.dot(hact, w_ref[...],
                           preferred_element_type=F32) * dis_ref[...]


def _tc_layer_call(acc, hn, dis, b, w, npad, h, hout, br=1024):
    g = npad // br
    return pl.pallas_call(
        _tc_layer_body,
        grid=(g,),
        in_specs=[
            pl.BlockSpec((NC, br, h), lambda i: (0, i, 0)),
            pl.BlockSpec((br, h), lambda i: (i, 0)),
            pl.BlockSpec((br, 1), lambda i: (i, 0)),
            pl.BlockSpec((1, h), lambda i: (0, 0)),
            pl.BlockSpec((h, hout), lambda i: (0, 0)),
        ],
        out_specs=pl.BlockSpec((br, hout), lambda i: (i, 0)),
        out_shape=jax.ShapeDtypeStruct((npad, hout), F32),
    )(acc, hn, dis, b, w)


def _tc_final_body(acc_ref, y_ref, dis_ref, b_ref, out_ref):
    out_ref[...] = ((acc_ref[0] + acc_ref[1] + y_ref[...]) * dis_ref[...]
                    + b_ref[...])


def _tc_final_call(acc3, y2d, dis2d, b3, rows):
    return pl.pallas_call(
        _tc_final_body,
        grid=(1,),
        in_specs=[
            pl.BlockSpec((NC, rows, 128), lambda i: (0, 0, 0)),
            pl.BlockSpec((rows, 128), lambda i: (0, 0)),
            pl.BlockSpec((rows, 128), lambda i: (0, 0)),
            pl.BlockSpec((1, 1), lambda i: (0, 0)),
        ],
        out_specs=pl.BlockSpec((rows, 128), lambda i: (0, 0)),
        out_shape=jax.ShapeDtypeStruct((rows, 128), F32),
    )(acc3, y2d, dis2d, b3)


def kernel(x, edge_index, W1, b1, W2, b2, W3, b3):
    n, d = x.shape
    h = W1.shape[1]
    e = edge_index.shape[1]

    npad = -(-(n + 1) // (NS * 128)) * (NS * 128)

    ntot = -(-e // (CHUNK * NS * 4)) * 4
    nsp0 = max(4, int(round(ntot * 0.73 / 4)) * 4)
    n0, n1 = nsp0, ntot - nsp0
    nmax = max(n0, n1)
    epad = NS * (n0 + n1) * CHUNK

    src = edge_index[0]
    dst = edge_index[1]
    padv = jnp.full((epad - e,), n, jnp.int32)

    def part(a):
        ch = jnp.concatenate([a, padv]).reshape(NS * (n0 + n1), CHUNK)
        c0 = ch[:NS * n0].reshape(NS, n0, CHUNK)
        c1 = ch[NS * n0:].reshape(NS, n1, CHUNK)
        fill = jnp.full((NS, nmax - min(n0, n1), CHUNK), n, jnp.int32)
        if n0 < n1:
            c0 = jnp.concatenate([c0, fill], axis=1)
        elif n1 < n0:
            c1 = jnp.concatenate([c1, fill], axis=1)
        return jnp.concatenate([c0, c1], axis=0)

    srcp3 = part(src)
    dstp = part(dst)
    srcp = srcp3.reshape(NW, nmax * CHUNK)
    xpad = jnp.pad(x, ((0, npad - n), (0, 0)))

    deg = _deg_call(dstp, npad, n0, n1)
    hn1, dis = _tc_prep_call(deg.reshape(NC, npad, 1), xpad, W1, npad, d, h)
    acc1 = _prop_call(hn1, srcp, dstp, npad, n0, n1, h)
    hn2 = _tc_layer_call(acc1, hn1, dis, b1.reshape(1, h), W2, npad, h, h)
    acc2 = _prop_call(hn2, srcp, dstp, npad, n0, n1, h)
    y = _tc_layer_call(acc2, hn2, dis, b2.reshape(1, h), W3, npad, h, 1)
    yf = y.reshape(npad)
    acc3 = _sprop_call(yf, srcp, dstp, npad, n0, n1)

    rows = npad // 128
    out2d = _tc_final_call(acc3.reshape(NC, rows, 128), yf.reshape(rows, 128),
                           dis.reshape(rows, 128), b3.reshape(1, 1), rows)
    return out2d.reshape(npad, 1)[:n]


# speedup vs baseline: 1.1809x; 1.1809x over previous
import functools

import jax
import jax.numpy as jnp
from jax import lax
from jax.experimental import pallas as pl
from jax.experimental.pallas import tpu as pltpu
from jax.experimental.pallas import tpu_sc as plsc

F32 = jnp.float32
NC, NS, L = 2, 16, 16
NW = NC * NS
CHUNK = 72


def _mesh():
    return plsc.VectorSubcoreMesh(core_axis_name="c", subcore_axis_name="s",
                                  num_cores=NC, num_subcores=NS)


def _worker_ids():
    c = lax.axis_index("c")
    s = lax.axis_index("s")
    return c, s, c * NS + s


def _fill(ref, start, n, value):
    v = jnp.full((L,), value, F32)
    for k in range(n // L):
        ref[pl.ds(start + k * L, L)] = v


def _deg_body(n0, n1, rps, dst_hbm, out_hbm, didx, ones, zb, acc):
    c, s, w = _worker_ids()
    nloc = jnp.where(c == 0, n0, n1)
    pltpu.sync_copy(dst_hbm.at[w], didx)
    _fill(ones, 0, CHUNK, 1.0)
    _fill(zb, 0, rps, 0.0)
    pltpu.sync_copy(zb, acc.at[pl.ds(s * rps, rps)])
    plsc.subcore_barrier()

    def body(j, carry):
        pltpu.sync_copy(ones, acc.at[didx.at[j]], add=True)
        return carry

    lax.fori_loop(0, nloc, body, 0)
    plsc.subcore_barrier()
    pltpu.sync_copy(acc.at[pl.ds(s * rps, rps)], out_hbm.at[c, pl.ds(s * rps, rps)])


def _deg_call(dstp, npad, n0, n1):
    rps = npad // NS
    nmax = max(n0, n1)
    fn = pl.kernel(
        functools.partial(_deg_body, n0, n1, rps),
        out_type=jax.ShapeDtypeStruct((NC, npad), F32),
        mesh=_mesh(),
        scratch_types=[
            pltpu.VMEM((nmax, CHUNK), jnp.int32),
            pltpu.VMEM((CHUNK,), F32),
            pltpu.VMEM((rps,), F32),
            pltpu.VMEM_SHARED((npad,), F32),
        ],
    )
    return fn(dstp)


def _prop_body(n0, n1, rps, d, hn_hbm, src_hbm, dst_hbm, out_hbm,
               sidx, dbuf, rbuf, acc,
               gs0, gs1, gs2, gs3, is0, is1, is2, is3):
    c, s, w = _worker_ids()
    nloc = jnp.where(c == 0, n0, n1)
    gsems = (gs0, gs1, gs2, gs3)
    isems = (is0, is1, is2, is3)
    pltpu.sync_copy(src_hbm.at[w], sidx)

    zv = jnp.zeros((L,), F32)
    zrows = CHUNK

    def zrow(r, carry):
        for k in range(d // L):
            rbuf[0, r, pl.ds(k * L, L)] = zv
        return carry

    lax.fori_loop(0, zrows, zrow, 0)
    for t in range(rps // zrows):
        pltpu.sync_copy(rbuf.at[0, pl.ds(0, zrows), :],
                        acc.at[pl.ds(s * rps + t * zrows, zrows), :])
    plsc.subcore_barrier()

    def dload(j, q):
        return pltpu.async_copy(dst_hbm.at[w, j], dbuf.at[q], isems[q])

    def dload_wait(j, q):
        pltpu.make_async_copy(dst_hbm.at[w, j], dbuf.at[q], isems[q]).wait()

    def gather(j, q):
        idx = sidx.at[pl.ds(j * CHUNK, CHUNK)]
        return pltpu.async_copy(hn_hbm.at[idx], rbuf.at[q], gsems[q])

    def gather_wait(j, q):
        idx = sidx.at[pl.ds(j * CHUNK, CHUNK)]
        pltpu.make_async_copy(hn_hbm.at[idx], rbuf.at[q], gsems[q]).wait()

    for q in range(4):
        dload(q, q)
        gather(q, q)

    def body(t, carry):
        jb = 4 * t
        for q in range(4):
            j = jb + q
            dload_wait(j, q)
            gather_wait(j, q)
            pltpu.sync_copy(rbuf.at[q], acc.at[dbuf.at[q]], add=True)

            @pl.when(j + 4 < nloc)
            def _():
                dload(j + 4, q)
                gather(j + 4, q)

        return carry

    lax.fori_loop(0, nloc // 4, body, 0)
    plsc.subcore_barrier()
    pltpu.sync_copy(acc.at[pl.ds(s * rps, rps), :],
                    out_hbm.at[c, pl.ds(s * rps, rps), :])


def _prop_call(hn, srcp, dstp, npad, n0, n1, d):
    rps = npad // NS
    nmax = max(n0, n1)
    fn = pl.kernel(
        functools.partial(_prop_body, n0, n1, rps, d),
        out_type=jax.ShapeDtypeStruct((NC, npad, d), F32),
        mesh=_mesh(),
        scratch_types=[
            pltpu.VMEM((nmax * CHUNK,), jnp.int32),
            pltpu.VMEM((4, CHUNK), jnp.int32),
            pltpu.VMEM((4, CHUNK, d), F32),
            pltpu.VMEM_SHARED((npad, d), F32),
        ] + [pltpu.SemaphoreType.DMA] * 8,
    )
    return fn(hn, srcp, dstp)


def _sprop_body(n0, n1, rps, y_hbm, src_hbm, dst_hbm, out_hbm,
                sidx, dbuf, vbuf, acc,
                gs0, gs1, gs2, gs3, is0, is1, is2, is3):
    c, s, w = _worker_ids()
    nloc = jnp.where(c == 0, n0, n1)
    gsems = (gs0, gs1, gs2, gs3)
    isems = (is0, is1, is2, is3)
    pltpu.sync_copy(src_hbm.at[w], sidx)
    _fill(vbuf.at[0], 0, CHUNK, 0.0)
    for t in range(rps // CHUNK):
        pltpu.sync_copy(vbuf.at[0], acc.at[pl.ds(s * rps + t * CHUNK, CHUNK)])
    plsc.subcore_barrier()

    def dload(j, q):
        return pltpu.async_copy(dst_hbm.at[w, j], dbuf.at[q], isems[q])

    def dload_wait(j, q):
        pltpu.make_async_copy(dst_hbm.at[w, j], dbuf.at[q], isems[q]).wait()

    def gather(j, q):
        idx = sidx.at[pl.ds(j * CHUNK, CHUNK)]
        return pltpu.async_copy(y_hbm.at[idx], vbuf.at[q], gsems[q])

    def gather_wait(j, q):
        idx = sidx.at[pl.ds(j * CHUNK, CHUNK)]
        pltpu.make_async_copy(y_hbm.at[idx], vbuf.at[q], gsems[q]).wait()

    for q in range(4):
        dload(q, q)
        gather(q, q)

    def body(t, carry):
        jb = 4 * t
        for q in range(4):
            j = jb + q
            dload_wait(j, q)
            gather_wait(j, q)
            pltpu.sync_copy(vbuf.at[q], acc.at[dbuf.at[q]], add=True)

            @pl.when(j + 4 < nloc)
            def _():
                dload(j + 4, q)
                gather(j + 4, q)

        return carry

    lax.fori_loop(0, nloc // 4, body, 0)
    plsc.subcore_barrier()
    pltpu.sync_copy(acc.at[pl.ds(s * rps, rps)], out_hbm.at[c, pl.ds(s * rps, rps)])


def _sprop_call(y, srcp, dstp, npad, n0, n1):
    rps = npad // NS
    nmax = max(n0, n1)
    fn = pl.kernel(
        functools.partial(_sprop_body, n0, n1, rps),
        out_type=jax.ShapeDtypeStruct((NC, npad), F32),
        mesh=_mesh(),
        scratch_types=[
            pltpu.VMEM((nmax * CHUNK,), jnp.int32),
            pltpu.VMEM((4, CHUNK), jnp.int32),
            pltpu.VMEM((4, CHUNK), F32),
            pltpu.VMEM_SHARED((npad,), F32),
        ] + [pltpu.SemaphoreType.DMA] * 8,
    )
    return fn(y, srcp, dstp)


def _tc_prep_body(deg_ref, x_ref, w_ref, hn_ref, dis_ref):
    dis = lax.rsqrt(deg_ref[0] + deg_ref[1] + 1.0)
    dis_ref[...] = dis
    hn_ref[...] = jnp.dot(x_ref[...], w_ref[...],
                          preferred_element_type=F32) * dis


def _tc_prep_call(deg3, xpad, w1, npad, d, h, br=1024):
    g = npad // br
    return pl.pallas_call(
        _tc_prep_body,
        grid=(g,),
        in_specs=[
            pl.BlockSpec((NC, br, 1), lambda i: (0, i, 0)),
            pl.BlockSpec((br, d), lambda i: (i, 0)),
            pl.BlockSpec((d, h), lambda i: (0, 0)),
        ],
        out_specs=[
            pl.BlockSpec((br, h), lambda i: (i, 0)),
            pl.BlockSpec((br, 1), lambda i: (i, 0)),
        ],
        out_shape=[
            jax.ShapeDtypeStruct((npad, h), F32),
            jax.ShapeDtypeStruct((npad, 1), F32),
        ],
    )(deg3, xpad, w1)


def _tc_layer_body(acc_ref, hn_ref, dis_ref, b_ref, w_ref, out_ref):
    a = acc_ref[0] + acc_ref[1] + hn_ref[...]
    hact = jnp.maximum(a * dis_ref[...] + b_ref[...], 0.0)
    out_ref[...] = jnp.dot(hact, w_ref[...],
                           preferred_element_type=F32) * dis_ref[...]


def _tc_layer_call(acc, hn, dis, b, w, npad, h, hout, br=1024):
    g = npad // br
    return pl.pallas_call(
        _tc_layer_body,
        grid=(g,),
        in_specs=[
            pl.BlockSpec((NC, br, h), lambda i: (0, i, 0)),
            pl.BlockSpec((br, h), lambda i: (i, 0)),
            pl.BlockSpec((br, 1), lambda i: (i, 0)),
            pl.BlockSpec((1, h), lambda i: (0, 0)),
            pl.BlockSpec((h, hout), lambda i: (0, 0)),
        ],
        out_specs=pl.BlockSpec((br, hout), lambda i: (i, 0)),
        out_shape=jax.ShapeDtypeStruct((npad, hout), F32),
    )(acc, hn, dis, b, w)


def _tc_final_body(acc_ref, y_ref, dis_ref, b_ref, out_ref):
    out_ref[...] = ((acc_ref[0] + acc_ref[1] + y_ref[...]) * dis_ref[...]
                    + b_ref[...])


def _tc_final_call(acc3, y2d, dis2d, b3, rows):
    return pl.pallas_call(
        _tc_final_body,
        grid=(1,),
        in_specs=[
            pl.BlockSpec((NC, rows, 128), lambda i: (0, 0, 0)),
            pl.BlockSpec((rows, 128), lambda i: (0, 0)),
            pl.BlockSpec((rows, 128), lambda i: (0, 0)),
            pl.BlockSpec((1, 1), lambda i: (0, 0)),
        ],
        out_specs=pl.BlockSpec((rows, 128), lambda i: (0, 0)),
        out_shape=jax.ShapeDtypeStruct((rows, 128), F32),
    )(acc3, y2d, dis2d, b3)


def kernel(x, edge_index, W1, b1, W2, b2, W3, b3):
    n, d = x.shape
    h = W1.shape[1]
    e = edge_index.shape[1]

    npad = -(-(n + 1) // (NS * 128)) * (NS * 128)

    ntot = -(-e // (CHUNK * NS * 4)) * 4
    nsp0 = ntot // 2 + (ntot // 2) % 4
    n0, n1 = nsp0, ntot - nsp0
    nmax = max(n0, n1)
    epad = NS * (n0 + n1) * CHUNK

    src = edge_index[0]
    dst = edge_index[1]
    padv = jnp.full((epad - e,), n, jnp.int32)

    def part(a):
        ch = jnp.concatenate([a, padv]).reshape(NS * (n0 + n1), CHUNK)
        c0 = ch[:NS * n0].reshape(NS, n0, CHUNK)
        c1 = ch[NS * n0:].reshape(NS, n1, CHUNK)
        fill = jnp.full((NS, nmax - min(n0, n1), CHUNK), n, jnp.int32)
        if n0 < n1:
            c0 = jnp.concatenate([c0, fill], axis=1)
        elif n1 < n0:
            c1 = jnp.concatenate([c1, fill], axis=1)
        return jnp.concatenate([c0, c1], axis=0)

    srcp3 = part(src)
    dstp = part(dst)
    srcp = srcp3.reshape(NW, nmax * CHUNK)
    xpad = jnp.pad(x, ((0, npad - n), (0, 0)))

    deg = _deg_call(dstp, npad, n0, n1)
    hn1, dis = _tc_prep_call(deg.reshape(NC, npad, 1), xpad, W1, npad, d, h)
    acc1 = _prop_call(hn1, srcp, dstp, npad, n0, n1, h)
    hn2 = _tc_layer_call(acc1, hn1, dis, b1.reshape(1, h), W2, npad, h, h)
    acc2 = _prop_call(hn2, srcp, dstp, npad, n0, n1, h)
    y = _tc_layer_call(acc2, hn2, dis, b2.reshape(1, h), W3, npad, h, 1)
    yf = y.reshape(npad)
    acc3 = _sprop_call(yf, srcp, dstp, npad, n0, n1)

    rows = npad // 128
    out2d = _tc_final_call(acc3.reshape(NC, rows, 128), yf.reshape(rows, 128),
                           dis.reshape(rows, 128), b3.reshape(1, 1), rows)
    return out2d.reshape(npad, 1)[:n]
